# baseline (speedup 1.0000x reference)
import functools

import jax
import jax.numpy as jnp
from jax import lax
from jax.experimental import pallas as pl
from jax.experimental.pallas import tpu as pltpu
from jax.experimental.pallas import tpu_sc as plsc

N = 10000
D = 128
K_ITERS = 3
LAMBDA_AMP = 0.1
LAM = LAMBDA_AMP / (2.0 * (1.0 - LAMBDA_AMP))

NC = 2
NS = 16
NW = NC * NS
DH = D // NC
CH = 128
NCH_T = 160
NCH_W = 80
EPAD = NS * NCH_T * CH
RPT = 640
NP = NS * RPT
SINK = N

_SC_MESH = plsc.VectorSubcoreMesh(
    core_axis_name="c", subcore_axis_name="s", num_cores=NC, num_subcores=NS)


@functools.partial(
    pl.kernel,
    out_type=jax.ShapeDtypeStruct((NC, NP // 2, D), jnp.float32),
    mesh=_SC_MESH,
    compiler_params=pltpu.CompilerParams(use_tc_tiling_on_sc=False),
    scratch_types=[
        pltpu.VMEM_SHARED((NP // 2, D), jnp.float32),
        pltpu.VMEM((NCH_T, CH), jnp.int32),
        pltpu.VMEM((NCH_T, CH), jnp.int32),
        [pltpu.VMEM((CH, D), jnp.float32) for _ in range(2)],
        [pltpu.SemaphoreType.DMA for _ in range(2)],
        [pltpu.SemaphoreType.DMA for _ in range(2)],
    ],
)
def _spmm_sc(xs_hbm, srcr_hbm, dstr_hbm, zeros_hbm, part_hbm,
             acc, idx_s, idx_d, rows, gsem, ssem):
    cid = lax.axis_index("c")
    sid = lax.axis_index("s")
    pltpu.sync_copy(srcr_hbm.at[sid], idx_s)
    pltpu.sync_copy(dstr_hbm.at[sid], idx_d)
    xs_c = xs_hbm
    pltpu.sync_copy(zeros_hbm, rows[0])
    for t in range(RPT // CH):
        pltpu.sync_copy(rows[0], acc.at[pl.ds(sid * RPT + t * CH, CH)])
    plsc.subcore_barrier()

    nbuf = 2
    nround = NCH_T // nbuf

    def body(j, _):
        for b in range(nbuf):
            c = jnp.int32(nbuf) * j + b

            @pl.when(j > 0)
            def _drain():
                pltpu.make_async_copy(
                    rows[b], acc.at[idx_d.at[c]], ssem[b]).wait()

            pltpu.async_copy(xs_c.at[idx_s.at[c]], rows[b], gsem[b])
        for b in range(nbuf):
            c = jnp.int32(nbuf) * j + b
            pltpu.make_async_copy(xs_c.at[idx_s.at[c]], rows[b],
                                  gsem[b]).wait()
            pltpu.async_copy(rows[b], acc.at[idx_d.at[c]], ssem[b],
                             add=True)
        return 0

    lax.fori_loop(jnp.int32(0), jnp.int32(nround), body, 0)
    for b in range(nbuf):
        pltpu.make_async_copy(
            rows[b], acc.at[idx_d.at[jnp.int32(b)]], ssem[b]).wait()
    plsc.subcore_barrier()
    for t in range(RPT // CH):
        sl = pl.ds(sid * RPT + t * CH, CH)
        pltpu.sync_copy(acc.at[sl], rows[0])
        pltpu.sync_copy(rows[0], part_hbm.at[cid, sl])


@functools.partial(
    pl.kernel,
    out_type=(jax.ShapeDtypeStruct((NC * NP,), jnp.float32),
              jax.ShapeDtypeStruct((NC * NP,), jnp.float32)),
    mesh=_SC_MESH,
    scratch_types=[
        pltpu.VMEM_SHARED((NP,), jnp.float32),
        pltpu.VMEM_SHARED((NP,), jnp.float32),
        pltpu.VMEM((NCH_W, CH), jnp.int32),
        pltpu.VMEM((NCH_W, CH), jnp.int32),
        pltpu.VMEM((CH,), jnp.float32),
        pltpu.VMEM((RPT,), jnp.float32),
    ],
)
def _deg_sc(srcr_hbm, dstr_hbm, dego_hbm, degi_hbm,
            dego, degi, idx_s, idx_d, ones_v, tmp_v):
    cid = lax.axis_index("c")
    sid = lax.axis_index("s")
    wid = cid * NS + sid
    pltpu.sync_copy(srcr_hbm.at[wid], idx_s)
    pltpu.sync_copy(dstr_hbm.at[wid], idx_d)
    for k in range(CH // 16):
        ones_v[pl.ds(16 * k, 16)] = jnp.full((16,), 1.0, jnp.float32)
    for k in range(RPT // 16):
        tmp_v[pl.ds(16 * k, 16)] = jnp.zeros((16,), jnp.float32)
    rows = pl.ds(sid * RPT, RPT)
    pltpu.sync_copy(tmp_v, dego.at[rows])
    pltpu.sync_copy(tmp_v, degi.at[rows])
    plsc.subcore_barrier()

    def body(j, _):
        pltpu.sync_copy(ones_v, dego.at[idx_s.at[j]], add=True)
        pltpu.sync_copy(ones_v, degi.at[idx_d.at[j]], add=True)
        return 0

    lax.fori_loop(jnp.int32(0), jnp.int32(NCH_W), body, 0)
    plsc.subcore_barrier()
    orows = pl.ds(cid * NP + sid * RPT, RPT)
    pltpu.sync_copy(dego.at[rows], tmp_v)
    pltpu.sync_copy(tmp_v, dego_hbm.at[orows])
    pltpu.sync_copy(degi.at[rows], tmp_v)
    pltpu.sync_copy(tmp_v, degi_hbm.at[orows])


def _norm_body(d0o_ref, d1o_ref, d0i_ref, d1i_ref, feat_ref,
               inv_in_ref, selfw_ref, inv_out_ref, xs_ref):
    inv_out = lax.rsqrt(d0o_ref[...] + d1o_ref[...] + 1.0)
    inv_in = lax.rsqrt(d0i_ref[...] + d1i_ref[...] + 1.0)
    inv_in_ref[...] = inv_in
    inv_out_ref[...] = inv_out
    selfw_ref[...] = inv_in * inv_out
    xs = feat_ref[...] * inv_out
    xs_ref[0, :, :] = xs[:, :DH]
    xs_ref[1, :, :] = xs[:, DH:]


def _norm_call(d0o, d1o, d0i, d1i, feat):
    return pl.pallas_call(
        _norm_body,
        out_shape=(jax.ShapeDtypeStruct((N, 1), jnp.float32),
                   jax.ShapeDtypeStruct((N, 1), jnp.float32),
                   jax.ShapeDtypeStruct((N, 1), jnp.float32),
                   jax.ShapeDtypeStruct((NC, N, DH), jnp.float32)),
    )(d0o, d1o, d0i, d1i, feat)


_RB = 2000


def _prox_body(part_ref, x_ref, feat_ref, inv_in_ref, selfw_ref, inv_out_ref,
               xn_ref, xsn_ref):
    p = jnp.concatenate([part_ref[0], part_ref[1]], axis=1)
    x = x_ref[...]
    feat = feat_ref[...]
    y = inv_in_ref[...] * p + selfw_ref[...] * x
    d = y - feat
    rn = jnp.sqrt(jnp.sum(d * d, axis=1, keepdims=True))
    score = jnp.maximum(rn - LAM, 0.0)
    safe = jnp.where(rn > 0.0, rn, 1.0)
    score = jnp.where(rn > 0.0, score / safe, score)
    xn = feat + score * d
    xn_ref[...] = xn
    xs = xn * inv_out_ref[...]
    xsn_ref[0, :, :] = xs[:, :DH]
    xsn_ref[1, :, :] = xs[:, DH:]


def _prox_call(part, x, feat, inv_in, selfw, inv_out):
    return pl.pallas_call(
        _prox_body,
        grid=(N // _RB,),
        in_specs=[pl.BlockSpec((NC, _RB, DH), lambda i: (i * 0, i, i * 0)),
                  pl.BlockSpec((_RB, D), lambda i: (i, i * 0)),
                  pl.BlockSpec((_RB, D), lambda i: (i, i * 0)),
                  pl.BlockSpec((_RB, 1), lambda i: (i, i * 0)),
                  pl.BlockSpec((_RB, 1), lambda i: (i, i * 0)),
                  pl.BlockSpec((_RB, 1), lambda i: (i, i * 0))],
        out_specs=(pl.BlockSpec((_RB, D), lambda i: (i, i * 0)),
                   pl.BlockSpec((NC, _RB, DH), lambda i: (i * 0, i, i * 0))),
        out_shape=(jax.ShapeDtypeStruct((N, D), jnp.float32),
                   jax.ShapeDtypeStruct((NC, N, DH), jnp.float32)),
    )(part, x, feat, inv_in, selfw, inv_out)


def kernel(feat, edge_index):
    src = edge_index[0].astype(jnp.int32)
    dst = edge_index[1].astype(jnp.int32)
    e = src.shape[0]
    pad = EPAD - e
    src_p = jnp.concatenate([src, jnp.zeros((pad,), jnp.int32)])
    dst_p = jnp.concatenate([dst, jnp.full((pad,), SINK, jnp.int32)])
    src_q = jnp.concatenate([src, jnp.full((pad,), SINK, jnp.int32)])
    srcr_t = src_p.reshape(NS, NCH_T, CH)
    dstr_t = (dst_p // 2).reshape(NS, NCH_T, CH)
    srcr_w = src_q.reshape(NW, NCH_W, CH)
    dstr_w = dst_p.reshape(NW, NCH_W, CH)
    zeros2 = jnp.zeros((CH, D), jnp.float32)

    dego_f, degi_f = _deg_sc(srcr_w, dstr_w)
    d0o = dego_f[:N].reshape(N, 1)
    d1o = dego_f[NP:NP + N].reshape(N, 1)
    d0i = degi_f[:N].reshape(N, 1)
    d1i = degi_f[NP:NP + N].reshape(N, 1)
    inv_in, selfw, inv_out, xs = _norm_call(d0o, d1o, d0i, d1i, feat)

    x = feat
    for _ in range(K_ITERS):
        xs_full = jnp.concatenate([xs[0], xs[1]], axis=1)
        part = _spmm_sc(xs_full, srcr_t, dstr_t, zeros2)
        fake = jnp.zeros((NC, N, DH), jnp.float32) + jnp.sum(part) * 1e-30
        x, xs = _prox_call(fake, x, feat, inv_in, selfw, inv_out)
    return x


# speedup vs baseline: 43.9422x; 43.9422x over previous
import functools

import jax
import jax.numpy as jnp
from jax import lax
from jax.experimental import pallas as pl
from jax.experimental.pallas import tpu as pltpu
from jax.experimental.pallas import tpu_sc as plsc

N = 10000
D = 128
K_ITERS = 3
LAMBDA_AMP = 0.1
LAM = LAMBDA_AMP / (2.0 * (1.0 - LAMBDA_AMP))

NC = 2
NS = 16
NW = NC * NS
DH = D // NC
CH = 128
NCH_T = 160
NCH_W = 80
EPAD = NS * NCH_T * CH
RPT = 640
NP = NS * RPT
SINK = N

_SC_MESH = plsc.VectorSubcoreMesh(
    core_axis_name="c", subcore_axis_name="s", num_cores=NC, num_subcores=NS)


@functools.partial(
    pl.kernel,
    out_type=jax.ShapeDtypeStruct((NC, NP, D), jnp.float32),
    mesh=_SC_MESH,
    scratch_types=[
        pltpu.VMEM_SHARED((NP, D), jnp.float32),
        pltpu.VMEM((NCH_W, CH), jnp.int32),
        pltpu.VMEM((NCH_W, CH), jnp.int32),
        [pltpu.VMEM((CH, D), jnp.float32) for _ in range(1)],
        [pltpu.SemaphoreType.DMA for _ in range(1)],
        [pltpu.SemaphoreType.DMA for _ in range(1)],
    ],
)
def _spmm_sc(xs_hbm, srcr_hbm, dstr_hbm, zeros_hbm, part_hbm,
             acc, idx_s, idx_d, rows, gsem, ssem):
    cid = lax.axis_index("c")
    sid = lax.axis_index("s")
    wid = cid * NS + sid
    pltpu.sync_copy(srcr_hbm.at[wid], idx_s)
    pltpu.sync_copy(dstr_hbm.at[wid], idx_d)
    xs_c = xs_hbm
    pltpu.sync_copy(zeros_hbm, rows[0])

    def zbody(t, _):
        pltpu.sync_copy(rows[0], acc.at[pl.ds(sid * RPT + t * CH, CH)])
        return 0

    lax.fori_loop(jnp.int32(0), jnp.int32(RPT // CH), zbody, 0)
    plsc.subcore_barrier()

    nbuf = 1
    nround = NCH_W // nbuf

    def body(j, _):
        for b in range(nbuf):
            c = jnp.int32(nbuf) * j + b

            @pl.when(j > 0)
            def _drain():
                pltpu.make_async_copy(
                    rows[b], acc.at[idx_d.at[c]], ssem[b]).wait()

            pltpu.async_copy(xs_c.at[idx_s.at[c]], rows[b], gsem[b])
        for b in range(nbuf):
            c = jnp.int32(nbuf) * j + b
            pltpu.make_async_copy(xs_c.at[idx_s.at[c]], rows[b],
                                  gsem[b]).wait()
            pltpu.async_copy(rows[b], acc.at[idx_d.at[c]], ssem[b],
                             add=True)
        return 0

    lax.fori_loop(jnp.int32(0), jnp.int32(nround), body, 0)
    for b in range(nbuf):
        pltpu.make_async_copy(
            rows[b], acc.at[idx_d.at[jnp.int32(b)]], ssem[b]).wait()
    plsc.subcore_barrier()

    def obody(t, _):
        sl = pl.ds(sid * RPT + t * CH, CH)
        pltpu.sync_copy(acc.at[sl], rows[0])
        pltpu.sync_copy(rows[0], part_hbm.at[cid, sl])
        return 0

    lax.fori_loop(jnp.int32(0), jnp.int32(RPT // CH), obody, 0)


@functools.partial(
    pl.kernel,
    out_type=(jax.ShapeDtypeStruct((NC * NP,), jnp.float32),
              jax.ShapeDtypeStruct((NC * NP,), jnp.float32)),
    mesh=_SC_MESH,
    scratch_types=[
        pltpu.VMEM_SHARED((NP,), jnp.float32),
        pltpu.VMEM_SHARED((NP,), jnp.float32),
        pltpu.VMEM((NCH_W, CH), jnp.int32),
        pltpu.VMEM((NCH_W, CH), jnp.int32),
        pltpu.VMEM((CH,), jnp.float32),
        pltpu.VMEM((RPT,), jnp.float32),
    ],
)
def _deg_sc(srcr_hbm, dstr_hbm, dego_hbm, degi_hbm,
            dego, degi, idx_s, idx_d, ones_v, tmp_v):
    cid = lax.axis_index("c")
    sid = lax.axis_index("s")
    wid = cid * NS + sid
    pltpu.sync_copy(srcr_hbm.at[wid], idx_s)
    pltpu.sync_copy(dstr_hbm.at[wid], idx_d)
    for k in range(CH // 16):
        ones_v[pl.ds(16 * k, 16)] = jnp.full((16,), 1.0, jnp.float32)
    for k in range(RPT // 16):
        tmp_v[pl.ds(16 * k, 16)] = jnp.zeros((16,), jnp.float32)
    rows = pl.ds(sid * RPT, RPT)
    pltpu.sync_copy(tmp_v, dego.at[rows])
    pltpu.sync_copy(tmp_v, degi.at[rows])
    plsc.subcore_barrier()

    def body(j, _):
        pltpu.sync_copy(ones_v, dego.at[idx_s.at[j]], add=True)
        pltpu.sync_copy(ones_v, degi.at[idx_d.at[j]], add=True)
        return 0

    lax.fori_loop(jnp.int32(0), jnp.int32(NCH_W), body, 0)
    plsc.subcore_barrier()
    orows = pl.ds(cid * NP + sid * RPT, RPT)
    pltpu.sync_copy(dego.at[rows], tmp_v)
    pltpu.sync_copy(tmp_v, dego_hbm.at[orows])
    pltpu.sync_copy(degi.at[rows], tmp_v)
    pltpu.sync_copy(tmp_v, degi_hbm.at[orows])


def _norm_body(d0o_ref, d1o_ref, d0i_ref, d1i_ref, feat_ref,
               inv_in_ref, selfw_ref, inv_out_ref, xs_ref):
    inv_out = lax.rsqrt(d0o_ref[...] + d1o_ref[...] + 1.0)
    inv_in = lax.rsqrt(d0i_ref[...] + d1i_ref[...] + 1.0)
    inv_in_ref[...] = inv_in
    inv_out_ref[...] = inv_out
    selfw_ref[...] = inv_in * inv_out
    xs_ref[...] = feat_ref[...] * inv_out


def _norm_call(d0o, d1o, d0i, d1i, feat):
    return pl.pallas_call(
        _norm_body,
        out_shape=(jax.ShapeDtypeStruct((N, 1), jnp.float32),
                   jax.ShapeDtypeStruct((N, 1), jnp.float32),
                   jax.ShapeDtypeStruct((N, 1), jnp.float32),
                   jax.ShapeDtypeStruct((N, D), jnp.float32)),
    )(d0o, d1o, d0i, d1i, feat)


_RB = 2000


def _prox_body(part_ref, x_ref, feat_ref, inv_in_ref, selfw_ref, inv_out_ref,
               xn_ref, xsn_ref):
    p = part_ref[0] + part_ref[1]
    x = x_ref[...]
    feat = feat_ref[...]
    y = inv_in_ref[...] * p + selfw_ref[...] * x
    d = y - feat
    rn = jnp.sqrt(jnp.sum(d * d, axis=1, keepdims=True))
    score = jnp.maximum(rn - LAM, 0.0)
    safe = jnp.where(rn > 0.0, rn, 1.0)
    score = jnp.where(rn > 0.0, score / safe, score)
    xn = feat + score * d
    xn_ref[...] = xn
    xsn_ref[...] = xn * inv_out_ref[...]


def _prox_call(part, x, feat, inv_in, selfw, inv_out):
    return pl.pallas_call(
        _prox_body,
        grid=(N // _RB,),
        in_specs=[pl.BlockSpec((NC, _RB, D), lambda i: (i * 0, i, i * 0)),
                  pl.BlockSpec((_RB, D), lambda i: (i, i * 0)),
                  pl.BlockSpec((_RB, D), lambda i: (i, i * 0)),
                  pl.BlockSpec((_RB, 1), lambda i: (i, i * 0)),
                  pl.BlockSpec((_RB, 1), lambda i: (i, i * 0)),
                  pl.BlockSpec((_RB, 1), lambda i: (i, i * 0))],
        out_specs=(pl.BlockSpec((_RB, D), lambda i: (i, i * 0)),
                   pl.BlockSpec((_RB, D), lambda i: (i, i * 0))),
        out_shape=(jax.ShapeDtypeStruct((N, D), jnp.float32),
                   jax.ShapeDtypeStruct((N, D), jnp.float32)),
    )(part, x, feat, inv_in, selfw, inv_out)


def kernel(feat, edge_index):
    src = edge_index[0].astype(jnp.int32)
    dst = edge_index[1].astype(jnp.int32)
    e = src.shape[0]
    pad = EPAD - e
    src_p = jnp.concatenate([src, jnp.zeros((pad,), jnp.int32)])
    dst_p = jnp.concatenate([dst, jnp.full((pad,), SINK, jnp.int32)])
    src_q = jnp.concatenate([src, jnp.full((pad,), SINK, jnp.int32)])
    srcr_t = src_p.reshape(NW, NCH_W, CH)
    dstr_t = dst_p.reshape(NW, NCH_W, CH)
    srcr_w = src_q.reshape(NW, NCH_W, CH)
    dstr_w = dst_p.reshape(NW, NCH_W, CH)
    zeros2 = jnp.zeros((CH, D), jnp.float32)

    dego_f, degi_f = _deg_sc(srcr_w, dstr_w)
    d0o = dego_f[:N].reshape(N, 1)
    d1o = dego_f[NP:NP + N].reshape(N, 1)
    d0i = degi_f[:N].reshape(N, 1)
    d1i = degi_f[NP:NP + N].reshape(N, 1)
    inv_in, selfw, inv_out, xs = _norm_call(d0o, d1o, d0i, d1i, feat)

    x = feat
    for _ in range(K_ITERS):
        part = _spmm_sc(xs, srcr_t, dstr_t, zeros2)
        x, xs = _prox_call(part, x, feat, inv_in, selfw, inv_out)
    return x


# speedup vs baseline: 64.3171x; 1.4637x over previous
import functools

import jax
import jax.numpy as jnp
from jax import lax
from jax.experimental import pallas as pl
from jax.experimental.pallas import tpu as pltpu
from jax.experimental.pallas import tpu_sc as plsc

N = 10000
D = 128
K_ITERS = 3
LAMBDA_AMP = 0.1
LAM = LAMBDA_AMP / (2.0 * (1.0 - LAMBDA_AMP))

NC = 2
NS = 16
NW = NC * NS
DH = D // NC
CH = 128
NCH_T = 160
NCH_W = 80
EPAD = NS * NCH_T * CH
RPT = 640
NP = NS * RPT
SINK = N

_SC_MESH = plsc.VectorSubcoreMesh(
    core_axis_name="c", subcore_axis_name="s", num_cores=NC, num_subcores=NS)


@functools.partial(
    pl.kernel,
    out_type=jax.ShapeDtypeStruct((NC, NP, DH), jnp.float32),
    mesh=_SC_MESH,
    compiler_params=pltpu.CompilerParams(use_tc_tiling_on_sc=False),
    scratch_types=[
        pltpu.VMEM_SHARED((NP, DH), jnp.float32),
        pltpu.VMEM((NCH_T, CH), jnp.int32),
        pltpu.VMEM((NCH_T, CH), jnp.int32),
        [pltpu.VMEM((CH, DH), jnp.float32) for _ in range(4)],
        [pltpu.SemaphoreType.DMA for _ in range(4)],
        [pltpu.SemaphoreType.DMA for _ in range(4)],
    ],
)
def _spmm_sc(xs_hbm, srcr_hbm, dstr_hbm, zeros_hbm, part_hbm,
             acc, idx_s, idx_d, rows, gsem, ssem):
    cid = lax.axis_index("c")
    sid = lax.axis_index("s")
    pltpu.sync_copy(srcr_hbm.at[sid], idx_s)
    pltpu.sync_copy(dstr_hbm.at[sid], idx_d)
    xs_c = xs_hbm.at[cid]
    pltpu.sync_copy(zeros_hbm, rows[0])
    for t in range(RPT // CH):
        pltpu.sync_copy(rows[0], acc.at[pl.ds(sid * RPT + t * CH, CH)])
    plsc.subcore_barrier()

    nbuf = 4
    nround = NCH_T // nbuf

    def body(j, _):
        for b in range(nbuf):
            c = jnp.int32(nbuf) * j + b

            @pl.when(j > 0)
            def _drain():
                pltpu.make_async_copy(
                    rows[b], acc.at[idx_d.at[c]], ssem[b]).wait()

            pltpu.async_copy(xs_c.at[idx_s.at[c]], rows[b], gsem[b])
        for b in range(nbuf):
            c = jnp.int32(nbuf) * j + b
            pltpu.make_async_copy(xs_c.at[idx_s.at[c]], rows[b],
                                  gsem[b]).wait()
            pltpu.async_copy(rows[b], acc.at[idx_d.at[c]], ssem[b],
                             add=True)
        return 0

    lax.fori_loop(jnp.int32(0), jnp.int32(nround), body, 0)
    for b in range(nbuf):
        pltpu.make_async_copy(
            rows[b], acc.at[idx_d.at[jnp.int32(b)]], ssem[b]).wait()
    plsc.subcore_barrier()
    for t in range(RPT // CH):
        sl = pl.ds(sid * RPT + t * CH, CH)
        pltpu.sync_copy(acc.at[sl], rows[0])
        pltpu.sync_copy(rows[0], part_hbm.at[cid, sl])


@functools.partial(
    pl.kernel,
    out_type=(jax.ShapeDtypeStruct((NC * NP,), jnp.float32),
              jax.ShapeDtypeStruct((NC * NP,), jnp.float32)),
    mesh=_SC_MESH,
    scratch_types=[
        pltpu.VMEM_SHARED((NP,), jnp.float32),
        pltpu.VMEM_SHARED((NP,), jnp.float32),
        pltpu.VMEM((NCH_W, CH), jnp.int32),
        pltpu.VMEM((NCH_W, CH), jnp.int32),
        pltpu.VMEM((CH,), jnp.float32),
        pltpu.VMEM((RPT,), jnp.float32),
    ],
)
def _deg_sc(srcr_hbm, dstr_hbm, dego_hbm, degi_hbm,
            dego, degi, idx_s, idx_d, ones_v, tmp_v):
    cid = lax.axis_index("c")
    sid = lax.axis_index("s")
    wid = cid * NS + sid
    pltpu.sync_copy(srcr_hbm.at[wid], idx_s)
    pltpu.sync_copy(dstr_hbm.at[wid], idx_d)
    for k in range(CH // 16):
        ones_v[pl.ds(16 * k, 16)] = jnp.full((16,), 1.0, jnp.float32)
    for k in range(RPT // 16):
        tmp_v[pl.ds(16 * k, 16)] = jnp.zeros((16,), jnp.float32)
    rows = pl.ds(sid * RPT, RPT)
    pltpu.sync_copy(tmp_v, dego.at[rows])
    pltpu.sync_copy(tmp_v, degi.at[rows])
    plsc.subcore_barrier()

    def body(j, _):
        pltpu.sync_copy(ones_v, dego.at[idx_s.at[j]], add=True)
        pltpu.sync_copy(ones_v, degi.at[idx_d.at[j]], add=True)
        return 0

    lax.fori_loop(jnp.int32(0), jnp.int32(NCH_W), body, 0)
    plsc.subcore_barrier()
    orows = pl.ds(cid * NP + sid * RPT, RPT)
    pltpu.sync_copy(dego.at[rows], tmp_v)
    pltpu.sync_copy(tmp_v, dego_hbm.at[orows])
    pltpu.sync_copy(degi.at[rows], tmp_v)
    pltpu.sync_copy(tmp_v, degi_hbm.at[orows])


def _norm_body(d0o_ref, d1o_ref, d0i_ref, d1i_ref, feat_ref,
               inv_in_ref, selfw_ref, inv_out_ref, xs_ref):
    inv_out = lax.rsqrt(d0o_ref[...] + d1o_ref[...] + 1.0)
    inv_in = lax.rsqrt(d0i_ref[...] + d1i_ref[...] + 1.0)
    inv_in_ref[...] = inv_in
    inv_out_ref[...] = inv_out
    selfw_ref[...] = inv_in * inv_out
    xs = feat_ref[...] * inv_out
    xs_ref[0, :, :] = xs[:, :DH]
    xs_ref[1, :, :] = xs[:, DH:]


def _norm_call(d0o, d1o, d0i, d1i, feat):
    return pl.pallas_call(
        _norm_body,
        out_shape=(jax.ShapeDtypeStruct((N, 1), jnp.float32),
                   jax.ShapeDtypeStruct((N, 1), jnp.float32),
                   jax.ShapeDtypeStruct((N, 1), jnp.float32),
                   jax.ShapeDtypeStruct((NC, N, DH), jnp.float32)),
    )(d0o, d1o, d0i, d1i, feat)


_RB = 2000


def _prox_body(part_ref, x_ref, feat_ref, inv_in_ref, selfw_ref, inv_out_ref,
               xn_ref, xsn_ref):
    p = jnp.concatenate([part_ref[0], part_ref[1]], axis=1)
    x = x_ref[...]
    feat = feat_ref[...]
    y = inv_in_ref[...] * p + selfw_ref[...] * x
    d = y - feat
    rn = jnp.sqrt(jnp.sum(d * d, axis=1, keepdims=True))
    score = jnp.maximum(rn - LAM, 0.0)
    safe = jnp.where(rn > 0.0, rn, 1.0)
    score = jnp.where(rn > 0.0, score / safe, score)
    xn = feat + score * d
    xn_ref[...] = xn
    xs = xn * inv_out_ref[...]
    xsn_ref[0, :, :] = xs[:, :DH]
    xsn_ref[1, :, :] = xs[:, DH:]


def _prox_call(part, x, feat, inv_in, selfw, inv_out):
    return pl.pallas_call(
        _prox_body,
        grid=(N // _RB,),
        in_specs=[pl.BlockSpec((NC, _RB, DH), lambda i: (i * 0, i, i * 0)),
                  pl.BlockSpec((_RB, D), lambda i: (i, i * 0)),
                  pl.BlockSpec((_RB, D), lambda i: (i, i * 0)),
                  pl.BlockSpec((_RB, 1), lambda i: (i, i * 0)),
                  pl.BlockSpec((_RB, 1), lambda i: (i, i * 0)),
                  pl.BlockSpec((_RB, 1), lambda i: (i, i * 0))],
        out_specs=(pl.BlockSpec((_RB, D), lambda i: (i, i * 0)),
                   pl.BlockSpec((NC, _RB, DH), lambda i: (i * 0, i, i * 0))),
        out_shape=(jax.ShapeDtypeStruct((N, D), jnp.float32),
                   jax.ShapeDtypeStruct((NC, N, DH), jnp.float32)),
    )(part, x, feat, inv_in, selfw, inv_out)


def kernel(feat, edge_index):
    src = edge_index[0].astype(jnp.int32)
    dst = edge_index[1].astype(jnp.int32)
    e = src.shape[0]
    pad = EPAD - e
    src_p = jnp.concatenate([src, jnp.zeros((pad,), jnp.int32)])
    dst_p = jnp.concatenate([dst, jnp.full((pad,), SINK, jnp.int32)])
    src_q = jnp.concatenate([src, jnp.full((pad,), SINK, jnp.int32)])
    srcr_t = src_p.reshape(NS, NCH_T, CH)
    dstr_t = dst_p.reshape(NS, NCH_T, CH)
    srcr_w = src_q.reshape(NW, NCH_W, CH)
    dstr_w = dst_p.reshape(NW, NCH_W, CH)
    zeros2 = jnp.zeros((CH, DH), jnp.float32)

    dego_f, degi_f = _deg_sc(srcr_w, dstr_w)
    d0o = dego_f[:N].reshape(N, 1)
    d1o = dego_f[NP:NP + N].reshape(N, 1)
    d0i = degi_f[:N].reshape(N, 1)
    d1i = degi_f[NP:NP + N].reshape(N, 1)
    inv_in, selfw, inv_out, xs = _norm_call(d0o, d1o, d0i, d1i, feat)

    x = feat
    for _ in range(K_ITERS):
        part = _spmm_sc(xs, srcr_t, dstr_t, zeros2)
        x, xs = _prox_call(part, x, feat, inv_in, selfw, inv_out)
    return x


# speedup vs baseline: 77.0150x; 1.1974x over previous
import functools

import jax
import jax.numpy as jnp
from jax import lax
from jax.experimental import pallas as pl
from jax.experimental.pallas import tpu as pltpu
from jax.experimental.pallas import tpu_sc as plsc

N = 10000
D = 128
K_ITERS = 3
LAMBDA_AMP = 0.1
LAM = LAMBDA_AMP / (2.0 * (1.0 - LAMBDA_AMP))

NC = 2
NS = 16
NW = NC * NS
DH = D // NC
CH = 128
NCH_T = 160
NCH_W = 80
EPAD = NS * NCH_T * CH
RPT = 640
NP = NS * RPT
SINK = N

_SC_MESH = plsc.VectorSubcoreMesh(
    core_axis_name="c", subcore_axis_name="s", num_cores=NC, num_subcores=NS)


@functools.partial(
    pl.kernel,
    out_type=jax.ShapeDtypeStruct((NC, NP, DH), jnp.float32),
    mesh=_SC_MESH,
    compiler_params=pltpu.CompilerParams(use_tc_tiling_on_sc=False),
    scratch_types=[
        pltpu.VMEM_SHARED((NP, DH), jnp.float32),
        pltpu.VMEM((NCH_T, CH), jnp.int32),
        pltpu.VMEM((NCH_T, CH), jnp.int32),
        [pltpu.VMEM((CH, DH), jnp.float32) for _ in range(4)],
        [pltpu.SemaphoreType.DMA for _ in range(4)],
        [pltpu.SemaphoreType.DMA for _ in range(4)],
    ],
)
def _spmm_sc(xs_hbm, srcr_hbm, dstr_hbm, zeros_hbm, part_hbm,
             acc, idx_s, idx_d, rows, gsem, ssem):
    cid = lax.axis_index("c")
    sid = lax.axis_index("s")
    pltpu.sync_copy(srcr_hbm.at[sid], idx_s)
    pltpu.sync_copy(dstr_hbm.at[sid], idx_d)
    xs_c = xs_hbm.at[cid]
    pltpu.sync_copy(zeros_hbm, rows[0])
    for t in range(RPT // CH):
        pltpu.sync_copy(rows[0], acc.at[pl.ds(sid * RPT + t * CH, CH)])
    plsc.subcore_barrier()

    nbuf = 4
    nround = NCH_T // nbuf

    def body(j, _):
        for b in range(nbuf):
            c = jnp.int32(nbuf) * j + b

            @pl.when(j > 0)
            def _drain():
                pltpu.make_async_copy(
                    rows[b], acc.at[idx_d.at[c]], ssem[b]).wait()

            pltpu.async_copy(xs_c.at[idx_s.at[c]], rows[b], gsem[b])
        for b in range(nbuf):
            c = jnp.int32(nbuf) * j + b
            pltpu.make_async_copy(xs_c.at[idx_s.at[c]], rows[b],
                                  gsem[b]).wait()
            pltpu.async_copy(rows[b], acc.at[idx_d.at[c]], ssem[b],
                             add=True)
        return 0

    lax.fori_loop(jnp.int32(0), jnp.int32(nround), body, 0)
    for b in range(nbuf):
        pltpu.make_async_copy(
            rows[b], acc.at[idx_d.at[jnp.int32(b)]], ssem[b]).wait()
    plsc.subcore_barrier()
    for t in range(RPT // CH):
        sl = pl.ds(sid * RPT + t * CH, CH)
        pltpu.sync_copy(acc.at[sl], rows[0])
        pltpu.sync_copy(rows[0], part_hbm.at[cid, sl])


@functools.partial(
    pl.kernel,
    out_type=(jax.ShapeDtypeStruct((NC * NP,), jnp.float32),
              jax.ShapeDtypeStruct((NC * NP,), jnp.float32)),
    mesh=_SC_MESH,
    scratch_types=[
        pltpu.VMEM_SHARED((NP,), jnp.float32),
        pltpu.VMEM_SHARED((NP,), jnp.float32),
        pltpu.VMEM((NCH_W, CH), jnp.int32),
        pltpu.VMEM((NCH_W, CH), jnp.int32),
        pltpu.VMEM((CH,), jnp.float32),
        pltpu.VMEM((RPT,), jnp.float32),
    ],
)
def _deg_sc(srcr_hbm, dstr_hbm, dego_hbm, degi_hbm,
            dego, degi, idx_s, idx_d, ones_v, tmp_v):
    cid = lax.axis_index("c")
    sid = lax.axis_index("s")
    wid = cid * NS + sid
    pltpu.sync_copy(srcr_hbm.at[wid], idx_s)
    pltpu.sync_copy(dstr_hbm.at[wid], idx_d)
    for k in range(CH // 16):
        ones_v[pl.ds(16 * k, 16)] = jnp.full((16,), 1.0, jnp.float32)
    for k in range(RPT // 16):
        tmp_v[pl.ds(16 * k, 16)] = jnp.zeros((16,), jnp.float32)
    rows = pl.ds(sid * RPT, RPT)
    pltpu.sync_copy(tmp_v, dego.at[rows])
    pltpu.sync_copy(tmp_v, degi.at[rows])
    plsc.subcore_barrier()

    def body(j, _):
        pltpu.sync_copy(ones_v, dego.at[idx_s.at[j]], add=True)
        pltpu.sync_copy(ones_v, degi.at[idx_d.at[j]], add=True)
        return 0

    lax.fori_loop(jnp.int32(0), jnp.int32(NCH_W), body, 0)
    plsc.subcore_barrier()
    orows = pl.ds(cid * NP + sid * RPT, RPT)
    pltpu.sync_copy(dego.at[rows], tmp_v)
    pltpu.sync_copy(tmp_v, dego_hbm.at[orows])
    pltpu.sync_copy(degi.at[rows], tmp_v)
    pltpu.sync_copy(tmp_v, degi_hbm.at[orows])


def _norm_body(d0o_ref, d1o_ref, d0i_ref, d1i_ref, feat_ref,
               inv_in_ref, selfw_ref, inv_out_ref, xs_ref):
    inv_out = lax.rsqrt(d0o_ref[...] + d1o_ref[...] + 1.0)
    inv_in = lax.rsqrt(d0i_ref[...] + d1i_ref[...] + 1.0)
    inv_in_ref[...] = inv_in
    inv_out_ref[...] = inv_out
    selfw_ref[...] = inv_in * inv_out
    xs = feat_ref[...] * inv_out
    xs_ref[0, :, :] = xs[:, :DH]
    xs_ref[1, :, :] = xs[:, DH:]


def _norm_call(d0o, d1o, d0i, d1i, feat):
    return pl.pallas_call(
        _norm_body,
        out_shape=(jax.ShapeDtypeStruct((N, 1), jnp.float32),
                   jax.ShapeDtypeStruct((N, 1), jnp.float32),
                   jax.ShapeDtypeStruct((N, 1), jnp.float32),
                   jax.ShapeDtypeStruct((NC, N, DH), jnp.float32)),
    )(d0o, d1o, d0i, d1i, feat)


_RB = 2000


def _prox_body(part_ref, x_ref, feat_ref, inv_in_ref, selfw_ref, inv_out_ref,
               xn_ref, xsn_ref):
    p = jnp.concatenate([part_ref[0], part_ref[1]], axis=1)
    x = x_ref[...]
    feat = feat_ref[...]
    y = inv_in_ref[...] * p + selfw_ref[...] * x
    d = y - feat
    rn = jnp.sqrt(jnp.sum(d * d, axis=1, keepdims=True))
    score = jnp.maximum(rn - LAM, 0.0)
    safe = jnp.where(rn > 0.0, rn, 1.0)
    score = jnp.where(rn > 0.0, score / safe, score)
    xn = feat + score * d
    xn_ref[...] = xn
    xs = xn * inv_out_ref[...]
    xsn_ref[0, :, :] = xs[:, :DH]
    xsn_ref[1, :, :] = xs[:, DH:]


def _prox_call(part, x, feat, inv_in, selfw, inv_out):
    return pl.pallas_call(
        _prox_body,
        grid=(N // _RB,),
        in_specs=[pl.BlockSpec((NC, _RB, DH), lambda i: (i * 0, i, i * 0)),
                  pl.BlockSpec((_RB, D), lambda i: (i, i * 0)),
                  pl.BlockSpec((_RB, D), lambda i: (i, i * 0)),
                  pl.BlockSpec((_RB, 1), lambda i: (i, i * 0)),
                  pl.BlockSpec((_RB, 1), lambda i: (i, i * 0)),
                  pl.BlockSpec((_RB, 1), lambda i: (i, i * 0))],
        out_specs=(pl.BlockSpec((_RB, D), lambda i: (i, i * 0)),
                   pl.BlockSpec((NC, _RB, DH), lambda i: (i * 0, i, i * 0))),
        out_shape=(jax.ShapeDtypeStruct((N, D), jnp.float32),
                   jax.ShapeDtypeStruct((NC, N, DH), jnp.float32)),
    )(part, x, feat, inv_in, selfw, inv_out)


def kernel(feat, edge_index):
    src = edge_index[0].astype(jnp.int32)
    dst = edge_index[1].astype(jnp.int32)
    e = src.shape[0]
    pad = EPAD - e
    def balance(ix, fill, groups, cap):
        per = e // groups
        body = ix.reshape(groups, per)
        tail = jnp.full((groups, cap - per), fill, jnp.int32)
        return jnp.concatenate([body, tail], axis=1)

    srcr_t = balance(src, 0, NS, NCH_T * CH).reshape(NS, NCH_T, CH)
    dstr_t = balance(dst, SINK, NS, NCH_T * CH).reshape(NS, NCH_T, CH)
    srcr_w = balance(src, SINK, NW, NCH_W * CH).reshape(NW, NCH_W, CH)
    dstr_w = balance(dst, SINK, NW, NCH_W * CH).reshape(NW, NCH_W, CH)
    zeros2 = jnp.zeros((CH, DH), jnp.float32)

    dego_f, degi_f = _deg_sc(srcr_w, dstr_w)
    d0o = dego_f[:N].reshape(N, 1)
    d1o = dego_f[NP:NP + N].reshape(N, 1)
    d0i = degi_f[:N].reshape(N, 1)
    d1i = degi_f[NP:NP + N].reshape(N, 1)
    inv_in, selfw, inv_out, xs = _norm_call(d0o, d1o, d0i, d1i, feat)

    x = feat
    for _ in range(K_ITERS):
        part = _spmm_sc(xs, srcr_t, dstr_t, zeros2)
        x, xs = _prox_call(part, x, feat, inv_in, selfw, inv_out)
    return x
